# baseline (speedup 1.0000x reference)
import functools

import jax
import jax.numpy as jnp
from jax import lax
from jax.experimental import pallas as pl
from jax.experimental.pallas import tpu as pltpu
from jax.experimental.pallas import tpu_sc as plsc

HALF = 128
NS = 16
EB = 80
SB = 25
ZR = 40
OW = 1000


@functools.lru_cache(maxsize=None)
def _sc_agg(N, E):
    PT = E // NS
    NB = PT // EB
    NSB = NB // SB
    NT = N // OW
    NZ = OW // ZR

    mesh = plsc.VectorSubcoreMesh(core_axis_name="c", subcore_axis_name="s")
    out_sds = jax.ShapeDtypeStruct((2, N, HALF), jnp.float32)

    @functools.partial(
        pl.kernel,
        out_type=[out_sds, out_sds],
        mesh=mesh,
        scratch_types=[
            pltpu.VMEM((2, SB, EB), jnp.int32),
            pltpu.VMEM((2, SB, EB), jnp.int32),
            pltpu.VMEM((2, SB, EB), jnp.float32),
            pltpu.VMEM((2, EB, HALF), jnp.float32),
            pltpu.VMEM((ZR, HALF), jnp.float32),
            pltpu.VMEM_SHARED((N, HALF), jnp.float32),
            pltpu.SemaphoreType.DMA((2,)),
            pltpu.SemaphoreType.DMA((2,)),
            pltpu.SemaphoreType.DMA((2,)),
        ],
    )
    def sc_agg(xs_hbm, src1_hbm, dst1_hbm, w1_hbm, src2_hbm, dst2_hbm, w2_hbm,
               out1_hbm, out2_hbm, srcb, dstb, wvb, rows, zerob, acc,
               gsem, ssem, isem):
        c = lax.axis_index("c")
        s = lax.axis_index("s")

        @pl.loop(0, ZR)
        def _zfill(r):
            zrow = zerob.at[r]
            for k in range(HALF // 16):
                zrow[pl.ds(k * 16, 16)] = jnp.zeros((16,), jnp.float32)

        def idx_trips(src_hbm, dst_hbm, w_hbm, sb, pp):
            return [
                (src_hbm.at[s].at[sb], srcb.at[pp], isem.at[pp]),
                (dst_hbm.at[s].at[sb], dstb.at[pp], isem.at[pp]),
                (w_hbm.at[s].at[sb], wvb.at[pp], isem.at[pp]),
            ]

        def idx_issue(src_hbm, dst_hbm, w_hbm, sb, pp):
            for t in idx_trips(src_hbm, dst_hbm, w_hbm, sb, pp):
                pltpu.async_copy(*t)

        def idx_drain(src_hbm, dst_hbm, w_hbm, sb, pp):
            for t in idx_trips(src_hbm, dst_hbm, w_hbm, sb, pp):
                pltpu.make_async_copy(*t).wait()

        def gather_trip(n, p):
            sb = n // SB
            return (xs_hbm.at[c].at[srcb.at[sb & 1].at[n - sb * SB]],
                    rows.at[p], gsem.at[p])

        def scatter_trip(n, p):
            sb = n // SB
            return (rows.at[p], acc.at[dstb.at[sb & 1].at[n - sb * SB]],
                    ssem.at[p])

        for conv, (src_hbm, dst_hbm, w_hbm, out_hbm) in enumerate([
                (src1_hbm, dst1_hbm, w1_hbm, out1_hbm),
                (src2_hbm, dst2_hbm, w2_hbm, out2_hbm)]):
            idx_issue(src_hbm, dst_hbm, w_hbm, 0, 0)
            idx_drain(src_hbm, dst_hbm, w_hbm, 0, 0)
            idx_issue(src_hbm, dst_hbm, w_hbm, 1, 1)

            @pl.when(s < NT)
            def _zero_stripe():
                @pl.loop(0, NZ)
                def _zero(j):
                    pltpu.sync_copy(zerob, acc.at[pl.ds(s * OW + j * ZR, ZR)])

            plsc.subcore_barrier()

            pltpu.async_copy(*gather_trip(0, 0))

            @pl.loop(0, NB)
            def _block(n):
                p = n & 1
                q = 1 - p
                i = n - (n // SB) * SB

                @pl.when(n > 0)
                def _():
                    pltpu.make_async_copy(*scatter_trip(n - 1, q)).wait()

                @pl.when((i == 0) & (n > 0) & (n + SB < NB))
                def _():
                    idx_issue(src_hbm, dst_hbm, w_hbm,
                              n // SB + 1, (n // SB + 1) & 1)

                @pl.when(n + 1 < NB)
                def _():
                    @pl.when(i == SB - 1)
                    def _():
                        idx_drain(src_hbm, dst_hbm, w_hbm,
                                  (n + 1) // SB, ((n + 1) // SB) & 1)
                    pltpu.async_copy(*gather_trip(n + 1, q))

                pltpu.make_async_copy(*gather_trip(n, p)).wait()
                wrow = wvb.at[(n // SB) & 1].at[i]

                @pl.loop(0, EB // 16)
                def _scale(g):
                    wv = wrow[pl.ds(g * 16, 16)]
                    for jj in range(16):
                        ws = wv[jj]
                        rrow = rows.at[p].at[g * 16 + jj]
                        for k in range(HALF // 16):
                            rrow[pl.ds(k * 16, 16)] = rrow[pl.ds(k * 16, 16)] * ws

                pltpu.async_copy(*scatter_trip(n, p), add=True)

            pltpu.make_async_copy(*scatter_trip(NB - 1, (NB - 1) & 1)).wait()

            plsc.subcore_barrier()

            @pl.when(s < NT)
            def _writeout():
                pltpu.sync_copy(acc.at[pl.ds(s * OW, OW)],
                                out_hbm.at[c].at[pl.ds(s * OW, OW)])

            plsc.subcore_barrier()

    return sc_agg


def _tc_x0_body(x_ref, w_ref, b_ref, o_ref):
    o_ref[...] = jnp.dot(x_ref[...], w_ref[...],
                         preferred_element_type=jnp.float32) + b_ref[...]


def _tc_conv_body(a1l_ref, a1r_ref, a2l_ref, a2r_ref, w1_ref, b1_ref,
                  w2_ref, b2_ref, x1_ref, x2_ref):
    w1t = w1_ref[0:HALF, :]
    w1b = w1_ref[HALF:2 * HALF, :]
    w2t = w2_ref[0:HALF, :]
    w2b = w2_ref[HALF:2 * HALF, :]
    x1_ref[...] = (jnp.dot(a1l_ref[...], w1t, preferred_element_type=jnp.float32)
                   + jnp.dot(a1r_ref[...], w1b, preferred_element_type=jnp.float32)
                   + b1_ref[...])
    x2_ref[...] = (jnp.dot(a2l_ref[...], w2t, preferred_element_type=jnp.float32)
                   + jnp.dot(a2r_ref[...], w2b, preferred_element_type=jnp.float32)
                   + b2_ref[...])


def kernel(x, edge_index, edge_weight, edge_index2, edge_weight2,
           W_ln, b_ln, W1, b1, W2, b2):
    N, D = x.shape
    E = edge_index.shape[1]
    BM = 1000
    grid = (N // BM,)

    PT = E // NS
    xs = jnp.stack([x[:, :HALF], x[:, HALF:]])
    esh = (NS, PT // (SB * EB), SB, EB)
    src1 = edge_index[0].astype(jnp.int32).reshape(esh)
    dst1 = edge_index[1].astype(jnp.int32).reshape(esh)
    src2 = edge_index2[0].astype(jnp.int32).reshape(esh)
    dst2 = edge_index2[1].astype(jnp.int32).reshape(esh)
    ew1 = edge_weight.reshape(esh)
    ew2 = edge_weight2.reshape(esh)

    x0 = pl.pallas_call(
        _tc_x0_body,
        grid=grid,
        in_specs=[
            pl.BlockSpec((BM, D), lambda i: (i, 0)),
            pl.BlockSpec((D, D), lambda i: (0, 0)),
            pl.BlockSpec((1, D), lambda i: (0, 0)),
        ],
        out_specs=pl.BlockSpec((BM, D), lambda i: (i, 0)),
        out_shape=jax.ShapeDtypeStruct((N, D), jnp.float32),
    )(x, W_ln, b_ln.reshape(1, D))

    agg1, agg2 = _sc_agg(N, E)(xs, src1, dst1, ew1, src2, dst2, ew2)

    half_spec = pl.BlockSpec((BM, HALF), lambda i: (i, 0))
    w_spec = pl.BlockSpec((D, D), lambda i: (0, 0))
    b_spec = pl.BlockSpec((1, D), lambda i: (0, 0))
    out_sds = jax.ShapeDtypeStruct((N, D), jnp.float32)
    x1, x2 = pl.pallas_call(
        _tc_conv_body,
        grid=grid,
        in_specs=[half_spec, half_spec, half_spec, half_spec,
                  w_spec, b_spec, w_spec, b_spec],
        out_specs=[pl.BlockSpec((BM, D), lambda i: (i, 0)),
                   pl.BlockSpec((BM, D), lambda i: (i, 0))],
        out_shape=[out_sds, out_sds],
    )(agg1[0], agg1[1], agg2[0], agg2[1],
      W1, b1.reshape(1, D), W2, b2.reshape(1, D))

    return x0, x1, x2


# speedup vs baseline: 1.6337x; 1.6337x over previous
import functools

import jax
import jax.numpy as jnp
import numpy as np
from jax import lax
from jax.experimental import pallas as pl
from jax.experimental.pallas import tpu as pltpu
from jax.experimental.pallas import tpu_sc as plsc

HALF = 128
NS = 16
EB = 80
SB = 25
ZR = 40
OW = 1000

_GDIMS = lax.GatherDimensionNumbers(offset_dims=(), collapsed_slice_dims=(0,),
                                    start_index_map=(0,))


def _splat(wv, jj):
    idx = lax.broadcast_in_dim(np.int32(jj), (16, 1), ())
    return lax.gather(wv, idx, _GDIMS, (1,),
                      mode=lax.GatherScatterMode.PROMISE_IN_BOUNDS)


@functools.lru_cache(maxsize=None)
def _sc_agg(N, E):
    PT = E // NS
    NB = PT // EB
    NSB = NB // SB
    NT = N // OW
    NZ = OW // ZR

    mesh = plsc.VectorSubcoreMesh(core_axis_name="c", subcore_axis_name="s")
    out_sds = jax.ShapeDtypeStruct((2, N, HALF), jnp.float32)

    @functools.partial(
        pl.kernel,
        out_type=[out_sds, out_sds],
        mesh=mesh,
        scratch_types=[
            pltpu.VMEM((2, SB, EB), jnp.int32),
            pltpu.VMEM((2, SB, EB), jnp.int32),
            pltpu.VMEM((2, SB, EB), jnp.float32),
            pltpu.VMEM((2, EB, HALF), jnp.float32),
            pltpu.VMEM((ZR, HALF), jnp.float32),
            pltpu.VMEM_SHARED((N, HALF), jnp.float32),
            pltpu.SemaphoreType.DMA((2,)),
            pltpu.SemaphoreType.DMA((2,)),
            pltpu.SemaphoreType.DMA((2,)),
        ],
    )
    def sc_agg(xs_hbm, src1_hbm, dst1_hbm, w1_hbm, src2_hbm, dst2_hbm, w2_hbm,
               out1_hbm, out2_hbm, srcb, dstb, wvb, rows, zerob, acc,
               gsem, ssem, isem):
        c = lax.axis_index("c")
        s = lax.axis_index("s")

        @pl.loop(0, ZR)
        def _zfill(r):
            zrow = zerob.at[r]
            for k in range(HALF // 16):
                zrow[pl.ds(k * 16, 16)] = jnp.zeros((16,), jnp.float32)

        def idx_trips(src_hbm, dst_hbm, w_hbm, sb, pp):
            return [
                (src_hbm.at[s].at[sb], srcb.at[pp], isem.at[pp]),
                (dst_hbm.at[s].at[sb], dstb.at[pp], isem.at[pp]),
                (w_hbm.at[s].at[sb], wvb.at[pp], isem.at[pp]),
            ]

        def idx_issue(src_hbm, dst_hbm, w_hbm, sb, pp):
            for t in idx_trips(src_hbm, dst_hbm, w_hbm, sb, pp):
                pltpu.async_copy(*t)

        def idx_drain(src_hbm, dst_hbm, w_hbm, sb, pp):
            for t in idx_trips(src_hbm, dst_hbm, w_hbm, sb, pp):
                pltpu.make_async_copy(*t).wait()

        def gather_trip(n, p):
            sb = n // SB
            return (xs_hbm.at[c].at[srcb.at[sb & 1].at[n - sb * SB]],
                    rows.at[p], gsem.at[p])

        def scatter_trip(n, p):
            sb = n // SB
            return (rows.at[p], acc.at[dstb.at[sb & 1].at[n - sb * SB]],
                    ssem.at[p])

        for conv, (src_hbm, dst_hbm, w_hbm, out_hbm) in enumerate([
                (src1_hbm, dst1_hbm, w1_hbm, out1_hbm),
                (src2_hbm, dst2_hbm, w2_hbm, out2_hbm)]):
            idx_issue(src_hbm, dst_hbm, w_hbm, 0, 0)
            idx_drain(src_hbm, dst_hbm, w_hbm, 0, 0)
            idx_issue(src_hbm, dst_hbm, w_hbm, 1, 1)

            @pl.when(s < NT)
            def _zero_stripe():
                @pl.loop(0, NZ)
                def _zero(j):
                    pltpu.sync_copy(zerob, acc.at[pl.ds(s * OW + j * ZR, ZR)])

            plsc.subcore_barrier()

            pltpu.async_copy(*gather_trip(0, 0))

            @pl.loop(0, NB)
            def _block(n):
                p = n & 1
                q = 1 - p
                i = n - (n // SB) * SB

                @pl.when(n > 0)
                def _():
                    pltpu.make_async_copy(*scatter_trip(n - 1, q)).wait()

                @pl.when((i == 0) & (n > 0) & (n + SB < NB))
                def _():
                    idx_issue(src_hbm, dst_hbm, w_hbm,
                              n // SB + 1, (n // SB + 1) & 1)

                @pl.when(n + 1 < NB)
                def _():
                    @pl.when(i == SB - 1)
                    def _():
                        idx_drain(src_hbm, dst_hbm, w_hbm,
                                  (n + 1) // SB, ((n + 1) // SB) & 1)
                    pltpu.async_copy(*gather_trip(n + 1, q))

                pltpu.make_async_copy(*gather_trip(n, p)).wait()
                wrow = wvb.at[(n // SB) & 1].at[i]

                rowsp = rows.at[p]

                @plsc.parallel_loop(0, EB // 16)
                def _scale(g):
                    wv = wrow[pl.ds(g * 16, 16)]
                    for jj in range(16):
                        wsplat = _splat(wv, jj)
                        rrow = rowsp.at[g * 16 + jj]
                        for k in range(HALF // 16):
                            rrow[pl.ds(k * 16, 16)] = (
                                rrow[pl.ds(k * 16, 16)] * wsplat)

                pltpu.async_copy(*scatter_trip(n, p), add=True)

            pltpu.make_async_copy(*scatter_trip(NB - 1, (NB - 1) & 1)).wait()

            plsc.subcore_barrier()

            @pl.when(s < NT)
            def _writeout():
                pltpu.sync_copy(acc.at[pl.ds(s * OW, OW)],
                                out_hbm.at[c].at[pl.ds(s * OW, OW)])

            plsc.subcore_barrier()

    return sc_agg


def _tc_x0_body(x_ref, w_ref, b_ref, o_ref):
    o_ref[...] = jnp.dot(x_ref[...], w_ref[...],
                         preferred_element_type=jnp.float32) + b_ref[...]


def _tc_conv_body(a1l_ref, a1r_ref, a2l_ref, a2r_ref, w1_ref, b1_ref,
                  w2_ref, b2_ref, x1_ref, x2_ref):
    w1t = w1_ref[0:HALF, :]
    w1b = w1_ref[HALF:2 * HALF, :]
    w2t = w2_ref[0:HALF, :]
    w2b = w2_ref[HALF:2 * HALF, :]
    x1_ref[...] = (jnp.dot(a1l_ref[...], w1t, preferred_element_type=jnp.float32)
                   + jnp.dot(a1r_ref[...], w1b, preferred_element_type=jnp.float32)
                   + b1_ref[...])
    x2_ref[...] = (jnp.dot(a2l_ref[...], w2t, preferred_element_type=jnp.float32)
                   + jnp.dot(a2r_ref[...], w2b, preferred_element_type=jnp.float32)
                   + b2_ref[...])


def kernel(x, edge_index, edge_weight, edge_index2, edge_weight2,
           W_ln, b_ln, W1, b1, W2, b2):
    N, D = x.shape
    E = edge_index.shape[1]
    BM = 1000
    grid = (N // BM,)

    PT = E // NS
    xs = jnp.stack([x[:, :HALF], x[:, HALF:]])
    esh = (NS, PT // (SB * EB), SB, EB)
    src1 = edge_index[0].astype(jnp.int32).reshape(esh)
    dst1 = edge_index[1].astype(jnp.int32).reshape(esh)
    src2 = edge_index2[0].astype(jnp.int32).reshape(esh)
    dst2 = edge_index2[1].astype(jnp.int32).reshape(esh)
    ew1 = edge_weight.reshape(esh)
    ew2 = edge_weight2.reshape(esh)

    x0 = pl.pallas_call(
        _tc_x0_body,
        grid=grid,
        in_specs=[
            pl.BlockSpec((BM, D), lambda i: (i, 0)),
            pl.BlockSpec((D, D), lambda i: (0, 0)),
            pl.BlockSpec((1, D), lambda i: (0, 0)),
        ],
        out_specs=pl.BlockSpec((BM, D), lambda i: (i, 0)),
        out_shape=jax.ShapeDtypeStruct((N, D), jnp.float32),
    )(x, W_ln, b_ln.reshape(1, D))

    agg1, agg2 = _sc_agg(N, E)(xs, src1, dst1, ew1, src2, dst2, ew2)

    half_spec = pl.BlockSpec((BM, HALF), lambda i: (i, 0))
    w_spec = pl.BlockSpec((D, D), lambda i: (0, 0))
    b_spec = pl.BlockSpec((1, D), lambda i: (0, 0))
    out_sds = jax.ShapeDtypeStruct((N, D), jnp.float32)
    x1, x2 = pl.pallas_call(
        _tc_conv_body,
        grid=grid,
        in_specs=[half_spec, half_spec, half_spec, half_spec,
                  w_spec, b_spec, w_spec, b_spec],
        out_specs=[pl.BlockSpec((BM, D), lambda i: (i, 0)),
                   pl.BlockSpec((BM, D), lambda i: (i, 0))],
        out_shape=[out_sds, out_sds],
    )(agg1[0], agg1[1], agg2[0], agg2[1],
      W1, b1.reshape(1, D), W2, b2.reshape(1, D))

    return x0, x1, x2
